# baseline (speedup 1.0000x reference)
import functools

import jax
import jax.numpy as jnp
from jax import lax
from jax.experimental import pallas as pl
from jax.experimental.pallas import tpu as pltpu
from jax.experimental.pallas import tpu_sc as plsc

NUM_TEAMS = 100000
EMBED_DIM = 128
BATCH = 16384
FEAT2 = 2 * EMBED_DIM
N_WORKERS = 32
ROWS_PER_W = BATCH // N_WORKERS
CHUNK = 128
N_CHUNKS = ROWS_PER_W // CHUNK
N_UNITS = 2 * N_CHUNKS
NBUF = 4
EPS = 1e-5
FEAT = FEAT2 + 1

N_TC_CHUNKS = 4

import numpy as _np
_p = _np.arange(FEAT2).reshape(-1, 2, 16)
PERM = _np.stack([_p[:, 0], _p[:, 1]], axis=-1).reshape(-1)

CROWS = BATCH // N_TC_CHUNKS
CROWS128 = CROWS // 128


def _sc_gather_body(table, idx1, idx2, feats, idx_v, bufs, bbufs, *sems):
    gsems = sems[:2]
    wsems = sems[2:]
    wid = lax.axis_index("s") * 2 + lax.axis_index("c")
    base = wid * ROWS_PER_W
    irow = wid * N_CHUNKS
    pltpu.sync_copy(idx1.at[pl.ds(irow, N_CHUNKS)], idx_v.at[pl.ds(0, N_CHUNKS)])
    pltpu.sync_copy(
        idx2.at[pl.ds(irow, N_CHUNKS)], idx_v.at[pl.ds(N_CHUNKS, N_CHUNKS)]
    )
    half_ulp = jnp.int32(0x8000)
    himask = jnp.int32(-65536)
    order = [0, 4, 1, 5, 2, 6, 3, 7]

    def to_bf16(u, slot):
        h, j = divmod(u, N_CHUNKS)
        cb = slot * CHUNK
        ob = j * CHUNK

        def row(r, carry):
            for k in range(EMBED_DIM // 32):
                a = lax.bitcast_convert_type(bufs[cb + r, pl.ds(k * 32, 16)], jnp.int32)
                bq = lax.bitcast_convert_type(
                    bufs[cb + r, pl.ds(k * 32 + 16, 16)], jnp.int32
                )
                lo = lax.shift_right_logical(a + half_ulp, 16)
                hi = (bq + half_ulp) & himask
                bbufs[ob + r, pl.ds(h * (EMBED_DIM // 2) + k * 16, 16)] = lo | hi
            return carry

        lax.fori_loop(0, CHUNK, row, 0)

    gathers = {}
    writes = {}
    for pos in range(2):
        u = order[pos]
        gathers[u] = pltpu.async_copy(
            table.at[idx_v.at[u]],
            bufs.at[pl.ds((pos % 2) * CHUNK, CHUNK)],
            gsems[pos % 2],
        )
    for pos, u in enumerate(order):
        gathers[u].wait()
        to_bf16(u, pos % 2)
        if pos + 2 < N_UNITS:
            nxt = order[pos + 2]
            gathers[nxt] = pltpu.async_copy(
                table.at[idx_v.at[nxt]],
                bufs.at[pl.ds((pos % 2) * CHUNK, CHUNK)],
                gsems[pos % 2],
            )
        if u >= N_CHUNKS:
            j = u - N_CHUNKS
            writes[j] = pltpu.async_copy(
                bbufs.at[pl.ds(j * CHUNK, CHUNK)],
                feats.at[pl.ds(base + j * CHUNK, CHUNK), :],
                wsems[j],
            )
    for j in range(N_CHUNKS):
        writes[j].wait()


_sc_gather = functools.partial(
    pl.kernel,
    mesh=plsc.VectorSubcoreMesh(core_axis_name="c", subcore_axis_name="s"),
    out_type=jax.ShapeDtypeStruct((BATCH, FEAT2 // 2), jnp.int32),
    scratch_types=[
        pltpu.VMEM((N_UNITS, CHUNK), jnp.int32),
        pltpu.VMEM((2 * CHUNK, EMBED_DIM), jnp.float32),
        pltpu.VMEM((ROWS_PER_W, FEAT2 // 2), jnp.int32),
    ]
    + [pltpu.SemaphoreType.DMA] * (2 + N_CHUNKS),
)(_sc_gather_body)


def _tc_bn_body(feats_hbm, s2d_ref, gw_ref, scal_ref, out_ref, x_ref, z_ref, sems):
    copies = []
    for i in range(N_TC_CHUNKS):
        cp = pltpu.make_async_copy(
            feats_hbm.at[pl.ds(i * CROWS128, CROWS128)],
            x_ref.at[pl.ds(i * CROWS128, CROWS128)],
            sems.at[i],
        )
        cp.start()
        copies.append(cp)
    gw = gw_ref[0, :]
    bconst = scal_ref[0, 0]
    ssum = jnp.zeros((FEAT2,), jnp.float32)
    ssq = jnp.zeros((FEAT2,), jnp.float32)
    for i in range(N_TC_CHUNKS):
        copies[i].wait()
        x = x_ref[pl.ds(i * CROWS128, CROWS128)].astype(jnp.float32)
        ssum = ssum + jnp.sum(jnp.sum(x, axis=0), axis=0)
        ssq = ssq + jnp.sum(jnp.sum(x * x, axis=0), axis=0)
    s = s2d_ref[...]
    inv_n = 1.0 / BATCH
    smean = jnp.sum(s) * inv_n
    svar = jnp.sum(s * s) * inv_n - smean * smean
    mean = ssum * inv_n
    var = ssq * inv_n - mean * mean
    c = gw[:FEAT2] * lax.rsqrt(var + EPS)
    cs = scal_ref[0, 1] * lax.rsqrt(svar + EPS)
    const = bconst - jnp.sum(c * mean) - cs * smean
    for i in range(N_TC_CHUNKS):
        x = x_ref[pl.ds(i * CROWS128, CROWS128)].astype(jnp.float32)
        z_ref[pl.ds(i * CROWS128, CROWS128), :] = jnp.sum(x * c, axis=2)
    zz = z_ref[...] + s * cs + const
    out_ref[...] = jax.nn.sigmoid(zz)


def _tc_bn(feats3, s2d, gw257, scal):
    return pl.pallas_call(
        _tc_bn_body,
        in_specs=[
            pl.BlockSpec(memory_space=pltpu.MemorySpace.HBM),
            pl.BlockSpec(memory_space=pltpu.VMEM),
            pl.BlockSpec(memory_space=pltpu.VMEM),
            pl.BlockSpec(memory_space=pltpu.SMEM),
        ],
        out_specs=pl.BlockSpec(memory_space=pltpu.VMEM),
        out_shape=jax.ShapeDtypeStruct((128, 128), jnp.float32),
        scratch_shapes=[
            pltpu.VMEM((128, 128, FEAT2), jnp.bfloat16),
            pltpu.VMEM((128, 128), jnp.float32),
            pltpu.SemaphoreType.DMA((N_TC_CHUNKS,)),
        ],
    )(feats3, s2d, gw257, scal)


def kernel(idsTensor, table, gamma, beta, W, b):
    idx1 = idsTensor[:, 0].astype(jnp.int32).reshape(128, 128)
    idx2 = idsTensor[:, 1].astype(jnp.int32).reshape(128, 128)
    s2d = idsTensor[:, 2].reshape(128, 128)
    w = W[0]
    gw = gamma * w
    gw257 = jnp.concatenate([gw[:FEAT2][PERM], gw[FEAT2:]]).reshape(1, FEAT + 0)
    scal = jnp.stack([b[0] + jnp.sum(beta * w), gw[FEAT2]]).reshape(1, 2)
    feats_u32 = _sc_gather(table, idx1, idx2)
    feats_bf = jax.lax.bitcast_convert_type(feats_u32, jnp.bfloat16)
    feats3 = feats_bf.reshape(128, 128, FEAT2)
    out = _tc_bn(feats3, s2d, gw257, scal)
    return out.reshape(BATCH, 1)


# speedup vs baseline: 2.6993x; 2.6993x over previous
import functools

import jax
import jax.numpy as jnp
from jax import lax
from jax.experimental import pallas as pl
from jax.experimental.pallas import tpu as pltpu
from jax.experimental.pallas import tpu_sc as plsc

NUM_TEAMS = 100000
EMBED_DIM = 128
BATCH = 16384
FEAT2 = 2 * EMBED_DIM
N_WORKERS = 32
ROWS_PER_W = BATCH // N_WORKERS
CHUNK = 128
N_CHUNKS = ROWS_PER_W // CHUNK
N_UNITS = 2 * N_CHUNKS
NBUF = 6
EPS = 1e-5

N_TC_CHUNKS = 4
CROWS = BATCH // N_TC_CHUNKS
CROWS128 = CROWS // 128


def _sc_gather_body(table, idx1, idx2, feats, idx_v, bufs, *sems):
    gsems = sems[:NBUF]
    wsems = sems[NBUF:]
    wid = lax.axis_index("s") * 2 + lax.axis_index("c")
    base = wid * ROWS_PER_W
    irow = wid * N_CHUNKS
    pltpu.sync_copy(idx1.at[pl.ds(irow, N_CHUNKS)], idx_v.at[pl.ds(0, N_CHUNKS)])
    pltpu.sync_copy(
        idx2.at[pl.ds(irow, N_CHUNKS)], idx_v.at[pl.ds(N_CHUNKS, N_CHUNKS)]
    )

    def buf_at(u):
        return bufs.at[pl.ds((u % NBUF) * CHUNK, CHUNK)]

    def feats_at(u):
        half, j = divmod(u, N_CHUNKS)
        return feats.at[
            pl.ds(base + j * CHUNK, CHUNK), pl.ds(half * EMBED_DIM, EMBED_DIM)
        ]

    gathers = {}
    writes = {}
    for u in range(min(NBUF, N_UNITS)):
        gathers[u] = pltpu.async_copy(table.at[idx_v.at[u]], buf_at(u), gsems[u % NBUF])
    for u in range(N_UNITS):
        gathers[u].wait()
        writes[u] = pltpu.async_copy(buf_at(u), feats_at(u), wsems[u % NBUF])
        if u + NBUF < N_UNITS:
            writes[u].wait()
            gathers[u + NBUF] = pltpu.async_copy(
                table.at[idx_v.at[u + NBUF]], buf_at(u + NBUF), gsems[(u + NBUF) % NBUF]
            )
    for u in range(max(0, N_UNITS - NBUF), N_UNITS):
        writes[u].wait()


_sc_gather = functools.partial(
    pl.kernel,
    mesh=plsc.VectorSubcoreMesh(core_axis_name="c", subcore_axis_name="s"),
    out_type=jax.ShapeDtypeStruct((BATCH, FEAT2), jnp.float32),
    scratch_types=[
        pltpu.VMEM((N_UNITS, CHUNK), jnp.int32),
        pltpu.VMEM((NBUF * CHUNK, EMBED_DIM), jnp.float32),
    ]
    + [pltpu.SemaphoreType.DMA] * (2 * NBUF),
)(_sc_gather_body)


def _tc_bn_body(feats_hbm, s2d_ref, gamma_ref, beta_ref, w_ref, b_ref, out_ref, x_ref, z_ref, sems):
    copies = []
    for i in range(N_TC_CHUNKS):
        cp = pltpu.make_async_copy(
            feats_hbm.at[pl.ds(i * CROWS128, CROWS128)],
            x_ref.at[pl.ds(i * CROWS128, CROWS128)],
            sems.at[i],
        )
        cp.start()
        copies.append(cp)
    wv = w_ref[0, :]
    gw = gamma_ref[...] * wv
    bconst = b_ref[0] + jnp.sum(beta_ref[...] * wv)
    ssum = jnp.zeros((FEAT2,), jnp.float32)
    ssq = jnp.zeros((FEAT2,), jnp.float32)
    for i in range(N_TC_CHUNKS):
        copies[i].wait()
        x = x_ref[pl.ds(i * CROWS128, CROWS128)]
        ssum = ssum + jnp.sum(jnp.sum(x, axis=0), axis=0)
        ssq = ssq + jnp.sum(jnp.sum(x * x, axis=0), axis=0)
    s = s2d_ref[...]
    inv_n = 1.0 / BATCH
    smean = jnp.sum(s) * inv_n
    svar = jnp.sum(s * s) * inv_n - smean * smean
    mean = ssum * inv_n
    var = ssq * inv_n - mean * mean
    c = gw[:FEAT2] * lax.rsqrt(var + EPS)
    cs = gw[FEAT2] * lax.rsqrt(svar + EPS)
    const = bconst - jnp.sum(c * mean) - cs * smean
    for i in range(N_TC_CHUNKS):
        x = x_ref[pl.ds(i * CROWS128, CROWS128)]
        z_ref[pl.ds(i * CROWS128, CROWS128), :] = jnp.sum(x * c, axis=2)
    zz = z_ref[...] + s * cs + const
    out_ref[...] = jax.nn.sigmoid(zz)


def _tc_bn(feats3, s2d, gamma, beta, W, b):
    return pl.pallas_call(
        _tc_bn_body,
        in_specs=[
            pl.BlockSpec(memory_space=pltpu.MemorySpace.HBM),
            pl.BlockSpec(memory_space=pltpu.VMEM),
            pl.BlockSpec(memory_space=pltpu.VMEM),
            pl.BlockSpec(memory_space=pltpu.VMEM),
            pl.BlockSpec(memory_space=pltpu.VMEM),
            pl.BlockSpec(memory_space=pltpu.VMEM),
        ],
        out_specs=pl.BlockSpec(memory_space=pltpu.VMEM),
        out_shape=jax.ShapeDtypeStruct((128, 128), jnp.float32),
        scratch_shapes=[
            pltpu.VMEM((128, 128, FEAT2), jnp.float32),
            pltpu.VMEM((128, 128), jnp.float32),
            pltpu.SemaphoreType.DMA((N_TC_CHUNKS,)),
        ],
    )(feats3, s2d, gamma, beta, W, b)


def kernel(idsTensor, table, gamma, beta, W, b):
    idx1 = idsTensor[:, 0].astype(jnp.int32).reshape(128, 128)
    idx2 = idsTensor[:, 1].astype(jnp.int32).reshape(128, 128)
    s2d = idsTensor[:, 2].reshape(128, 128)
    feats = _sc_gather(table, idx1, idx2)
    feats3 = feats.reshape(128, 128, FEAT2)
    out = _tc_bn(feats3, s2d, gamma, beta, W, b)
    return out.reshape(BATCH, 1)
